# baseline (speedup 1.0000x reference)
import functools

import jax
import jax.numpy as jnp
from jax import lax
from jax.experimental import pallas as pl
from jax.experimental.pallas import tpu as pltpu
from jax.experimental.pallas import tpu_sc as plsc

NC = 2
NS = 16
NW = NC * NS

V = 1000000
NB = 7813
SR = NB * 64
NBUF = 4

NT = 50
NBATCH = 16384
BPW = NBATCH // NW
NBLK = NT * 4

_mesh = plsc.VectorSubcoreMesh(core_axis_name="c", subcore_axis_name="s")
_params = pltpu.CompilerParams(use_tc_tiling_on_sc=True, needs_layout_passes=False)


@functools.partial(
    pl.kernel,
    out_type=jax.ShapeDtypeStruct((SR, 128), jnp.float32),
    mesh=_mesh,
    scratch_types=[
        [pltpu.VMEM((64, 128), jnp.float32)] * NBUF,
        [pltpu.VMEM((64, 128), jnp.float32)] * NBUF,
        pltpu.SemaphoreType.DMA,
        pltpu.SemaphoreType.DMA,
    ],
    compiler_params=_params,
)
def _phase1(lut_t, scr, gbuf, wbuf, gsem, wsem):
    c = lax.axis_index("c")
    s = lax.axis_index("s")
    w = s * NC + c
    iota = lax.iota(jnp.int32, 16)

    def fire(m, gb):
        k = m * NW + w

        @pl.when(k < NB)
        def _():
            pltpu.async_copy(lut_t.at[:, pl.ds(k * 128, 128)], gb, gsem)

    for pre in range(NBUF - 1):
        fire(pre, gbuf[pre])

    def body(m, carry):
        for b4 in range(NBUF):
            mm = m * NBUF + b4
            k = mm * NW + w
            gb, wb = gbuf[b4], wbuf[b4]

            @pl.when(k < NB)
            def _():
                pltpu.make_async_copy(scr.at[pl.ds(0, 64)], gb, gsem).wait()

            fire(mm + NBUF - 1, gbuf[(b4 + NBUF - 1) % NBUF])

            @pl.when(k < NB)
            def _():
                @pl.when(mm >= NBUF)
                def _():
                    pltpu.make_async_copy(scr.at[pl.ds(0, 64)], wb, wsem).wait()

                @plsc.parallel_loop(0, 32, step=1, unroll=2)
                def _(p):
                    h = p & 1
                    dvec = 16 * ((p >> 1) & 3) + iota
                    j0 = 16 * (p >> 3)
                    for kk in range(16):
                        jrot = j0 + ((iota + kk) & 15)
                        vals = plsc.load_gather(gb, [dvec, 2 * jrot + h]) * 8.0
                        plsc.store_scatter(wb, [jrot, dvec + 64 * h], vals)

                pltpu.async_copy(wb, scr.at[pl.ds(k * 64, 64)], wsem)

        return carry

    lax.fori_loop(0, (NB + NW * NBUF - 1) // (NW * NBUF), body, 0)
    for b4 in range(NBUF):
        pltpu.make_async_copy(scr.at[pl.ds(0, 64)], wbuf[b4], wsem).wait()


@functools.partial(
    pl.kernel,
    out_type=jax.ShapeDtypeStruct((NT, 64, NBATCH), jnp.float32),
    mesh=_mesh,
    scratch_types=[
        pltpu.VMEM((NT, BPW), jnp.int32),
        [pltpu.VMEM((128, 128), jnp.float32)] * NBUF,
        [pltpu.VMEM((64, 128), jnp.float32)] * NBUF,
        [pltpu.VMEM((128,), jnp.int32)] * NBUF,
        [pltpu.VMEM((128,), jnp.int32)] * NBUF,
        pltpu.SemaphoreType.DMA,
        pltpu.SemaphoreType.DMA,
    ],
    compiler_params=_params,
)
def _phase2(scr, xt, out, idx_v, gbuf, wbuf, hbuf, pbuf, gsem, wsem):
    c = lax.axis_index("c")
    s = lax.axis_index("s")
    w = s * NC + c
    bbase = w * BPW
    pltpu.sync_copy(xt.at[:, pl.ds(bbase, BPW)], idx_v)
    iota = lax.iota(jnp.int32, 16)

    def prep_and_fire(n, hb, pb, gb):
        t = n >> 2
        j = n & 3

        def qbody(q, carry):
            v = idx_v[t, pl.ds(j * 128 + 16 * q, 16)]
            hb[pl.ds(16 * q, 16)] = lax.shift_right_logical(v, 1)
            pb[pl.ds(16 * q, 16)] = (v & 1) * 64
            return carry

        lax.fori_loop(0, 8, qbody, 0, unroll=8)
        pltpu.async_copy(scr.at[hb], gb, gsem)

    for pre in range(NBUF - 1):
        prep_and_fire(pre, hbuf[pre], pbuf[pre], gbuf[pre])

    def outer(i, carry):
        for b4 in range(NBUF):
            n = i * NBUF + b4
            t = n >> 2
            j = n & 3
            gb, wb, pb = gbuf[b4], wbuf[b4], pbuf[b4]
            pltpu.make_async_copy(scr.at[pl.ds(0, 128)], gb, gsem).wait()

            @pl.when(n + NBUF - 1 < NBLK)
            def _():
                prep_and_fire(n + NBUF - 1, hbuf[(b4 + NBUF - 1) % NBUF],
                              pbuf[(b4 + NBUF - 1) % NBUF],
                              gbuf[(b4 + NBUF - 1) % NBUF])

            @pl.when(n >= NBUF)
            def _():
                pltpu.make_async_copy(out.at[0, :, pl.ds(0, 128)], wb, wsem).wait()

            @plsc.parallel_loop(0, 8, step=1, unroll=1)
            def _(q):
                b0 = 16 * q
                for kk in range(16):
                    bvec = b0 + ((iota + kk) & 15)
                    parv = plsc.load_gather(pb, [bvec])
                    for m4 in range(4):
                        dvec = 16 * m4 + iota
                        vals = plsc.load_gather(gb, [bvec, dvec + parv])
                        plsc.store_scatter(wb, [dvec, bvec], vals)

            pltpu.async_copy(wb, out.at[t, :, pl.ds(bbase + j * 128, 128)], wsem)
        return carry

    lax.fori_loop(0, NBLK // NBUF, outer, 0)
    for b4 in range(NBUF):
        pltpu.make_async_copy(out.at[0, :, pl.ds(0, 128)], wbuf[b4], wsem).wait()


def kernel(x, lut):
    scr = _phase1(lut.T)
    outp = _phase2(scr, x.astype(jnp.int32).T)
    return outp.transpose(2, 0, 1)


# speedup vs baseline: 1.3812x; 1.3812x over previous
import functools

import jax
import jax.numpy as jnp
from jax import lax
from jax.experimental import pallas as pl
from jax.experimental.pallas import tpu as pltpu
from jax.experimental.pallas import tpu_sc as plsc

NC = 2
NS = 16
NW = NC * NS

V = 1000000
NB = 7813
SR = NB * 64
NBUF = 4

NT = 50
NBATCH = 16384
BPW = NBATCH // NW
NBLK = NT * 4

_mesh = plsc.VectorSubcoreMesh(core_axis_name="c", subcore_axis_name="s")
_params = pltpu.CompilerParams(use_tc_tiling_on_sc=True, needs_layout_passes=False)


@functools.partial(
    pl.kernel,
    out_type=jax.ShapeDtypeStruct((SR, 128), jnp.float32),
    mesh=_mesh,
    scratch_types=[
        [pltpu.VMEM((64, 128), jnp.float32)] * NBUF,
        [pltpu.VMEM((64, 128), jnp.float32)] * NBUF,
        pltpu.SemaphoreType.DMA,
        pltpu.SemaphoreType.DMA,
    ],
    compiler_params=_params,
)
def _phase1(lut_t, scr, gbuf, wbuf, gsem, wsem):
    c = lax.axis_index("c")
    s = lax.axis_index("s")
    w = s * NC + c
    iota = lax.iota(jnp.int32, 16)

    def fire(m, gb):
        k = m * NW + w

        @pl.when(k < NB)
        def _():
            pltpu.async_copy(lut_t.at[:, pl.ds(k * 128, 128)], gb, gsem)

    for pre in range(NBUF - 1):
        fire(pre, gbuf[pre])

    def body(m, carry):
        for b4 in range(NBUF):
            mm = m * NBUF + b4
            k = mm * NW + w
            gb, wb = gbuf[b4], wbuf[b4]

            @pl.when(k < NB)
            def _():
                pltpu.make_async_copy(scr.at[pl.ds(0, 64)], gb, gsem).wait()

            fire(mm + NBUF - 1, gbuf[(b4 + NBUF - 1) % NBUF])

            @pl.when(k < NB)
            def _():
                @pl.when(mm >= NBUF)
                def _():
                    pltpu.make_async_copy(scr.at[pl.ds(0, 64)], wb, wsem).wait()

                @plsc.parallel_loop(0, 32, step=1, unroll=2)
                def _(p):
                    h = p & 1
                    dvec = 16 * ((p >> 1) & 3) + iota
                    j0 = 16 * (p >> 3)
                    for kk in range(16):
                        jrot = j0 + ((iota + kk) & 15)
                        vals = plsc.load_gather(gb, [dvec, 2 * jrot + h]) * 8.0
                        plsc.store_scatter(wb, [jrot, dvec + 64 * h], vals)

                pltpu.async_copy(wb, scr.at[pl.ds(k * 64, 64)], wsem)

        return carry

    lax.fori_loop(0, (NB + NW * NBUF - 1) // (NW * NBUF), body, 0)
    for b4 in range(NBUF):
        pltpu.make_async_copy(scr.at[pl.ds(0, 64)], wbuf[b4], wsem).wait()


@functools.partial(
    pl.kernel,
    out_type=jax.ShapeDtypeStruct((NT, 64, NBATCH), jnp.float32),
    mesh=_mesh,
    scratch_types=[
        pltpu.VMEM((NT, BPW), jnp.int32),
        [pltpu.VMEM((128, 128), jnp.float32)] * NBUF,
        [pltpu.VMEM((64, 128), jnp.float32)] * NBUF,
        [pltpu.VMEM((128,), jnp.int32)] * NBUF,
        [pltpu.VMEM((128,), jnp.int32)] * NBUF,
        pltpu.SemaphoreType.DMA,
        pltpu.SemaphoreType.DMA,
    ],
    compiler_params=_params,
)
def _phase2(scr, xt, out, idx_v, gbuf, wbuf, hbuf, pbuf, gsem, wsem):
    c = lax.axis_index("c")
    s = lax.axis_index("s")
    w = s * NC + c
    bbase = w * BPW
    pltpu.sync_copy(xt.at[:, pl.ds(bbase, BPW)], idx_v)
    iota = lax.iota(jnp.int32, 16)

    def prep_and_fire(n, hb, pb, gb):
        t = n >> 2
        j = n & 3

        def qbody(q, carry):
            v = idx_v[t, pl.ds(j * 128 + 16 * q, 16)]
            hb[pl.ds(16 * q, 16)] = lax.shift_right_logical(v, 1)
            pb[pl.ds(16 * q, 16)] = (v & 1) * 64
            return carry

        lax.fori_loop(0, 8, qbody, 0, unroll=8)
        pltpu.async_copy(scr.at[hb], gb, gsem)

    for pre in range(NBUF - 1):
        prep_and_fire(pre, hbuf[pre], pbuf[pre], gbuf[pre])

    def outer(i, carry):
        for b4 in range(NBUF):
            n = i * NBUF + b4
            t = n >> 2
            j = n & 3
            gb, wb, pb = gbuf[b4], wbuf[b4], pbuf[b4]
            pltpu.make_async_copy(scr.at[pl.ds(0, 128)], gb, gsem).wait()

            @pl.when(n + NBUF - 1 < NBLK)
            def _():
                prep_and_fire(n + NBUF - 1, hbuf[(b4 + NBUF - 1) % NBUF],
                              pbuf[(b4 + NBUF - 1) % NBUF],
                              gbuf[(b4 + NBUF - 1) % NBUF])

            @pl.when(n >= NBUF)
            def _():
                pltpu.make_async_copy(out.at[0, :, pl.ds(0, 128)], wb, wsem).wait()

            @plsc.parallel_loop(0, 32, step=1, unroll=1)
            def _(p):
                dvec = 16 * (p >> 3) + iota
                b0 = 16 * (p & 7)
                for kk in range(16):
                    bvec = b0 + ((iota + kk) & 15)
                    parv = plsc.load_gather(pb, [bvec])
                    vals = plsc.load_gather(gb, [bvec, dvec + parv])
                    plsc.store_scatter(wb, [dvec, bvec], vals)

            pltpu.async_copy(wb, out.at[t, :, pl.ds(bbase + j * 128, 128)], wsem)
        return carry

    lax.fori_loop(0, NBLK // NBUF, outer, 0)
    for b4 in range(NBUF):
        pltpu.make_async_copy(out.at[0, :, pl.ds(0, 128)], wbuf[b4], wsem).wait()


def kernel(x, lut):
    scr = _phase1(lut.T)
    outp = _phase2(scr, x.astype(jnp.int32).T)
    return outp.transpose(2, 0, 1)
